# baseline (speedup 1.0000x reference)
import jax
import jax.numpy as jnp
from jax import lax
from jax.experimental import pallas as pl
from jax.experimental.pallas import tpu as pltpu
from jax.experimental.pallas import tpu_sc as plsc

TABLE_ROWS = 16384
NUM_WORKERS = 32
TOTAL = 128 * 8192
CHUNK = 8192
NUM_CHUNKS = TOTAL // NUM_WORKERS // CHUNK
INNER = CHUNK // 16
UNROLL = 16


def _sc_body(idx_hbm, table_hbm, out_hbm, table_v, idx_v, out_v,
             sem_tab, sem_i0, sem_i1, sem_o0, sem_o1):
    c = lax.axis_index("c")
    s = lax.axis_index("s")
    wid = s * 2 + c
    rh = wid // 2
    th_w = (wid % 2) * 32

    tab_cp = pltpu.async_copy(table_hbm, table_v, sem_tab)
    pltpu.async_copy(idx_hbm.at[pl.ds(rh, 1), pl.ds(th_w, 8), :, :],
                     idx_v.at[pl.ds(0, 1)], sem_i0)
    pltpu.async_copy(idx_hbm.at[pl.ds(rh, 1), pl.ds(th_w + 8, 8), :, :],
                     idx_v.at[pl.ds(1, 1)], sem_i1)
    tab_cp.wait()

    def chunk_step(ci, carry):
        par = ci & 1
        th0 = th_w + ci * 8

        @pl.when(par == 0)
        def _():
            pltpu.make_async_copy(idx_hbm.at[pl.ds(0, 1), pl.ds(0, 8), :, :],
                                  idx_v.at[pl.ds(0, 1)], sem_i0).wait()

        @pl.when(par == 1)
        def _():
            pltpu.make_async_copy(idx_hbm.at[pl.ds(0, 1), pl.ds(0, 8), :, :],
                                  idx_v.at[pl.ds(1, 1)], sem_i1).wait()

        @pl.when(jnp.logical_and(ci >= 2, par == 0))
        def _():
            pltpu.make_async_copy(out_hbm.at[pl.ds(0, 8), pl.ds(0, 16), :],
                                  out_v.at[pl.ds(0, 8)], sem_o0).wait()

        @pl.when(jnp.logical_and(ci >= 2, par == 1))
        def _():
            pltpu.make_async_copy(out_hbm.at[pl.ds(0, 8), pl.ds(0, 16), :],
                                  out_v.at[pl.ds(8, 8)], sem_o1).wait()

        ob = par * 8

        @plsc.parallel_loop(0, INNER, unroll=UNROLL)
        def body(i):
            th_i = i >> 6
            rl = (i >> 3) & 7
            j = i & 7
            a = idx_v[par, th_i, rl, pl.ds(j * 16, 16)]
            ax = a + (a & jnp.int32(-128))
            x = plsc.load_gather(table_v, [ax])
            y = plsc.load_gather(table_v, [ax | 128])
            out_v[ob + rl, 2 * th_i, pl.ds(j * 16, 16)] = x
            out_v[ob + rl, 2 * th_i + 1, pl.ds(j * 16, 16)] = y

        dst = out_hbm.at[pl.ds(8 * rh, 8), pl.ds(2 * th0, 16), :]

        @pl.when(par == 0)
        def _():
            pltpu.async_copy(out_v.at[pl.ds(0, 8)], dst, sem_o0)

        @pl.when(par == 1)
        def _():
            pltpu.async_copy(out_v.at[pl.ds(8, 8)], dst, sem_o1)

        nxt_th = jnp.minimum(th0 + 16, 56)
        nxt = idx_hbm.at[pl.ds(rh, 1), pl.ds(nxt_th, 8), :, :]

        @pl.when(jnp.logical_and(ci + 2 < NUM_CHUNKS, par == 0))
        def _():
            pltpu.async_copy(nxt, idx_v.at[pl.ds(0, 1)], sem_i0)

        @pl.when(jnp.logical_and(ci + 2 < NUM_CHUNKS, par == 1))
        def _():
            pltpu.async_copy(nxt, idx_v.at[pl.ds(1, 1)], sem_i1)

        return carry

    lax.fori_loop(0, NUM_CHUNKS, chunk_step, 0)

    pltpu.make_async_copy(out_hbm.at[pl.ds(0, 8), pl.ds(0, 16), :],
                          out_v.at[pl.ds(0, 8)], sem_o0).wait()
    pltpu.make_async_copy(out_hbm.at[pl.ds(0, 8), pl.ds(0, 16), :],
                          out_v.at[pl.ds(8, 8)], sem_o1).wait()


@jax.jit
def _run(idx_p, table_p):
    mesh = plsc.VectorSubcoreMesh(core_axis_name="c", subcore_axis_name="s")
    fn = pl.kernel(
        _sc_body,
        mesh=mesh,
        out_type=jax.ShapeDtypeStruct((128, 128, 128), jnp.float32),
        scratch_types=[
            pltpu.VMEM((2 * TABLE_ROWS,), jnp.float32),
            pltpu.VMEM((2, 8, 8, 128), jnp.int32),
            pltpu.VMEM((16, 16, 128), jnp.float32),
            pltpu.SemaphoreType.DMA,
            pltpu.SemaphoreType.DMA,
            pltpu.SemaphoreType.DMA,
            pltpu.SemaphoreType.DMA,
            pltpu.SemaphoreType.DMA,
        ],
        compiler_params=pltpu.CompilerParams(needs_layout_passes=False),
    )
    return fn(idx_p, table_p)


def kernel(indices, hilbert_coords):
    idx_p = (indices.astype(jnp.int32)
             .reshape(16, 8, 64, 128).transpose(0, 2, 1, 3))
    table_p = (hilbert_coords.astype(jnp.float32)
               .reshape(128, 128, 2).transpose(0, 2, 1).reshape(-1))
    out_p = _run(idx_p, table_p)
    return (out_p.reshape(128, 64, 2, 128).transpose(0, 1, 3, 2)
            .reshape(128, 8192, 2))


# speedup vs baseline: 1.0096x; 1.0096x over previous
import jax
import jax.numpy as jnp
from jax import lax
from jax.experimental import pallas as pl
from jax.experimental.pallas import tpu as pltpu
from jax.experimental.pallas import tpu_sc as plsc

TABLE_ROWS = 16384
NUM_WORKERS = 32
TOTAL = 128 * 8192
CHUNK = 8192
NUM_CHUNKS = TOTAL // NUM_WORKERS // CHUNK
INNER = CHUNK // 16
UNROLL = 8


def _sc_body(idx_hbm, table_hbm, out_hbm, table_v, idx_v, out_v,
             sem_tab, sem_i0, sem_i1, sem_o0, sem_o1):
    c = lax.axis_index("c")
    s = lax.axis_index("s")
    wid = s * 2 + c
    rh = wid // 2
    th_w = (wid % 2) * 32

    tab_cp = pltpu.async_copy(table_hbm, table_v, sem_tab)
    pltpu.async_copy(idx_hbm.at[pl.ds(rh, 1), pl.ds(th_w, 8), :, :],
                     idx_v.at[pl.ds(0, 1)], sem_i0)
    pltpu.async_copy(idx_hbm.at[pl.ds(rh, 1), pl.ds(th_w + 8, 8), :, :],
                     idx_v.at[pl.ds(1, 1)], sem_i1)
    tab_cp.wait()

    def chunk_step(ci, carry):
        par = ci & 1
        th0 = th_w + ci * 8

        @pl.when(par == 0)
        def _():
            pltpu.make_async_copy(idx_hbm.at[pl.ds(0, 1), pl.ds(0, 8), :, :],
                                  idx_v.at[pl.ds(0, 1)], sem_i0).wait()

        @pl.when(par == 1)
        def _():
            pltpu.make_async_copy(idx_hbm.at[pl.ds(0, 1), pl.ds(0, 8), :, :],
                                  idx_v.at[pl.ds(1, 1)], sem_i1).wait()

        @pl.when(jnp.logical_and(ci >= 2, par == 0))
        def _():
            pltpu.make_async_copy(out_hbm.at[pl.ds(0, 8), pl.ds(0, 16), :],
                                  out_v.at[pl.ds(0, 8)], sem_o0).wait()

        @pl.when(jnp.logical_and(ci >= 2, par == 1))
        def _():
            pltpu.make_async_copy(out_hbm.at[pl.ds(0, 8), pl.ds(0, 16), :],
                                  out_v.at[pl.ds(8, 8)], sem_o1).wait()

        ob = par * 8

        @plsc.parallel_loop(0, INNER, unroll=UNROLL)
        def body(i):
            th_i = i >> 6
            rl = (i >> 3) & 7
            j = i & 7
            a = idx_v[par, th_i, rl, pl.ds(j * 16, 16)]
            ax = a + (a & jnp.int32(-128))
            x = plsc.load_gather(table_v, [ax])
            y = plsc.load_gather(table_v, [ax | 128])
            out_v[ob + rl, 2 * th_i, pl.ds(j * 16, 16)] = x
            out_v[ob + rl, 2 * th_i + 1, pl.ds(j * 16, 16)] = y

        dst = out_hbm.at[pl.ds(8 * rh, 8), pl.ds(2 * th0, 16), :]

        @pl.when(par == 0)
        def _():
            pltpu.async_copy(out_v.at[pl.ds(0, 8)], dst, sem_o0)

        @pl.when(par == 1)
        def _():
            pltpu.async_copy(out_v.at[pl.ds(8, 8)], dst, sem_o1)

        nxt_th = jnp.minimum(th0 + 16, 56)
        nxt = idx_hbm.at[pl.ds(rh, 1), pl.ds(nxt_th, 8), :, :]

        @pl.when(jnp.logical_and(ci + 2 < NUM_CHUNKS, par == 0))
        def _():
            pltpu.async_copy(nxt, idx_v.at[pl.ds(0, 1)], sem_i0)

        @pl.when(jnp.logical_and(ci + 2 < NUM_CHUNKS, par == 1))
        def _():
            pltpu.async_copy(nxt, idx_v.at[pl.ds(1, 1)], sem_i1)

        return carry

    lax.fori_loop(0, NUM_CHUNKS, chunk_step, 0)

    pltpu.make_async_copy(out_hbm.at[pl.ds(0, 8), pl.ds(0, 16), :],
                          out_v.at[pl.ds(0, 8)], sem_o0).wait()
    pltpu.make_async_copy(out_hbm.at[pl.ds(0, 8), pl.ds(0, 16), :],
                          out_v.at[pl.ds(8, 8)], sem_o1).wait()


@jax.jit
def _run(idx_p, table_p):
    mesh = plsc.VectorSubcoreMesh(core_axis_name="c", subcore_axis_name="s")
    fn = pl.kernel(
        _sc_body,
        mesh=mesh,
        out_type=jax.ShapeDtypeStruct((128, 128, 128), jnp.float32),
        scratch_types=[
            pltpu.VMEM((2 * TABLE_ROWS,), jnp.float32),
            pltpu.VMEM((2, 8, 8, 128), jnp.int32),
            pltpu.VMEM((16, 16, 128), jnp.float32),
            pltpu.SemaphoreType.DMA,
            pltpu.SemaphoreType.DMA,
            pltpu.SemaphoreType.DMA,
            pltpu.SemaphoreType.DMA,
            pltpu.SemaphoreType.DMA,
        ],
        compiler_params=pltpu.CompilerParams(needs_layout_passes=False),
    )
    return fn(idx_p, table_p)


def kernel(indices, hilbert_coords):
    idx_p = (indices.astype(jnp.int32)
             .reshape(16, 8, 64, 128).transpose(0, 2, 1, 3))
    table_p = (hilbert_coords.astype(jnp.float32)
               .reshape(128, 128, 2).transpose(0, 2, 1).reshape(-1))
    out_p = _run(idx_p, table_p)
    return (out_p.reshape(128, 64, 2, 128).transpose(0, 1, 3, 2)
            .reshape(128, 8192, 2))
